# baseline (speedup 1.0000x reference)
import jax
import jax.numpy as jnp
from jax.experimental import pallas as pl
from jax.experimental.pallas import tpu as pltpu

N = 4096
BM = 128


def _body(q_ref, at_ref, xt_ref, yt_ref, c_ref, out_ref, gmax_ref, cmax_ref):
    i = pl.program_id(0)
    qx = jnp.sum(q_ref[...] * xt_ref[...], axis=1)
    aty = jnp.sum(at_ref[...] * yt_ref[...], axis=1)
    ct = c_ref[0, pl.ds(i * BM, BM)]
    pg = qx + aty + ct
    m = jnp.max(jnp.abs(pg))

    @pl.when(i == 0)
    def _init():
        gmax_ref[0, 0] = m
        cmax_ref[0, 0] = jnp.max(jnp.abs(c_ref[...]))

    @pl.when(i > 0)
    def _acc():
        gmax_ref[0, 0] = jnp.maximum(gmax_ref[0, 0], m)

    @pl.when(i == pl.num_programs(0) - 1)
    def _fin():
        out_ref[0, 0] = gmax_ref[0, 0] / (1.0 + cmax_ref[0, 0])


def kernel(Q, AT, b, c, x, y, Iy, il, iu, l, u):
    xt = x.reshape(1, N)
    yt = y.reshape(1, N)
    crow = c.reshape(1, N)
    grid = N // BM
    out = pl.pallas_call(
        _body,
        grid=(grid,),
        in_specs=[
            pl.BlockSpec((BM, N), lambda i: (i, 0)),
            pl.BlockSpec((BM, N), lambda i: (i, 0)),
            pl.BlockSpec((1, N), lambda i: (0, 0)),
            pl.BlockSpec((1, N), lambda i: (0, 0)),
            pl.BlockSpec((1, N), lambda i: (0, 0)),
        ],
        out_specs=pl.BlockSpec(memory_space=pltpu.SMEM),
        out_shape=jax.ShapeDtypeStruct((1, 1), jnp.float32),
        scratch_shapes=[
            pltpu.SMEM((1, 1), jnp.float32),
            pltpu.SMEM((1, 1), jnp.float32),
        ],
    )(Q, AT, xt, yt, crow)
    return out[0, 0]


# speedup vs baseline: 1.1903x; 1.1903x over previous
import jax
import jax.numpy as jnp
from jax.experimental import pallas as pl
from jax.experimental.pallas import tpu as pltpu

N = 4096
H = N // 2
BM = 256


def _body(q1_ref, q2_ref, a1_ref, a2_ref, xt_ref, yt_ref, c_ref, out_ref,
          gmax_ref, cmax_ref):
    i = pl.program_id(0)
    s = (jnp.sum(q1_ref[...] * xt_ref[0, :H], axis=1)
         + jnp.sum(q2_ref[...] * xt_ref[0, H:], axis=1)
         + jnp.sum(a1_ref[...] * yt_ref[0, :H], axis=1)
         + jnp.sum(a2_ref[...] * yt_ref[0, H:], axis=1))
    pg = s + c_ref[0, pl.ds(i * BM, BM)]
    m = jnp.max(jnp.abs(pg))

    @pl.when(i == 0)
    def _init():
        gmax_ref[0, 0] = m
        cmax_ref[0, 0] = jnp.max(jnp.abs(c_ref[...]))

    @pl.when(i > 0)
    def _acc():
        gmax_ref[0, 0] = jnp.maximum(gmax_ref[0, 0], m)

    @pl.when(i == pl.num_programs(0) - 1)
    def _fin():
        out_ref[0, 0] = gmax_ref[0, 0] / (1.0 + cmax_ref[0, 0])


def kernel(Q, AT, b, c, x, y, Iy, il, iu, l, u):
    xt = x.reshape(1, N)
    yt = y.reshape(1, N)
    crow = c.reshape(1, N)
    grid = N // BM
    out = pl.pallas_call(
        _body,
        grid=(grid,),
        in_specs=[
            pl.BlockSpec((BM, H), lambda i: (i, 0)),
            pl.BlockSpec((BM, H), lambda i: (i, 1)),
            pl.BlockSpec((BM, H), lambda i: (i, 0)),
            pl.BlockSpec((BM, H), lambda i: (i, 1)),
            pl.BlockSpec((1, N), lambda i: (0, 0)),
            pl.BlockSpec((1, N), lambda i: (0, 0)),
            pl.BlockSpec((1, N), lambda i: (0, 0)),
        ],
        out_specs=pl.BlockSpec(memory_space=pltpu.SMEM),
        out_shape=jax.ShapeDtypeStruct((1, 1), jnp.float32),
        scratch_shapes=[
            pltpu.SMEM((1, 1), jnp.float32),
            pltpu.SMEM((1, 1), jnp.float32),
        ],
    )(Q, Q, AT, AT, xt, yt, crow)
    return out[0, 0]
